# baseline (speedup 1.0000x reference)
import jax
import jax.numpy as jnp
from jax import lax
from jax.experimental import pallas as pl
from jax.experimental.pallas import tpu as pltpu
from jax.experimental.pallas import tpu_sc as plsc

N_EMBED = 768
CLF_TOKEN = 0
ROWS = 16384
SEQ = 2048
NC, NS = 2, 16
NW = NC * NS
RWS = 256
SC_ROWS = NW * RWS
RT = ROWS - SC_ROWS
C = 64
NCH = RWS // C
NP = NCH // 2
LANES = 16
NJ = N_EMBED // LANES
BR = 2048


def _sc_body(h_hbm, x_hbm, w_hbm, b_hbm, out_hbm,
             hb0, hb1, wv, xv, bv, ov, s0, s1):
  cid = lax.axis_index("c")
  sid = lax.axis_index("s")
  wid = sid * NC + cid
  gbase = RT + wid * RWS
  bidx = gbase // SEQ
  roff = gbase % SEQ

  pltpu.sync_copy(w_hbm, wv)
  pltpu.sync_copy(b_hbm, bv)
  pltpu.sync_copy(x_hbm.at[bidx, pl.ds(roff, RWS)], xv)
  pltpu.async_copy(h_hbm.at[bidx, pl.ds(roff, C), :], hb0, s0)

  lane = lax.broadcasted_iota(jnp.int32, (LANES,), 0)
  perms = [lane ^ sh for sh in (8, 4, 2, 1)]

  def lane_sum(v):
    for perm in perms:
      v = v + jnp.take_along_axis(v, perm, axis=0, mode="promise_in_bounds")
    return v

  zero16 = jnp.zeros((LANES,), jnp.float32)

  def compute_chunk(buf, chunk_idx):
    def group(gi, carry):
      rr = gi * LANES
      @plsc.parallel_loop(0, NJ, unroll=2,
                          carry=tuple(zero16 for _ in range(LANES)))
      def accs(j, acc):
        off = j * LANES
        wj = wv[pl.ds(off, LANES)]
        return tuple(
            acc[i] + buf[rr + i, pl.ds(off, LANES)] * wj
            for i in range(LANES))

      out_vec = lane_sum(accs[0])
      for i in range(1, LANES):
        out_vec = jnp.where(lane == i, lane_sum(accs[i]), out_vec)
      rloc = chunk_idx * C + rr
      tokv = xv[pl.ds(rloc, LANES)]
      maskv = jnp.where(tokv == CLF_TOKEN, jnp.float32(1.0),
                        jnp.float32(0.0))
      res = out_vec * maskv + bv[...]
      ov[pl.ds(rloc, LANES)] = res
      return carry

    lax.fori_loop(0, C // LANES, group, 0)

  def start(chunk, buf, sem):
    pltpu.async_copy(h_hbm.at[bidx, pl.ds(roff + chunk * C, C), :], buf, sem)

  def wait(chunk, buf, sem):
    pltpu.make_async_copy(h_hbm.at[bidx, pl.ds(roff + chunk * C, C), :], buf,
                          sem).wait()

  for p in range(NP):
    c0 = p * 2
    start(c0 + 1, hb1, s1)
    wait(c0, hb0, s0)
    compute_chunk(hb0, c0)
    if p < NP - 1:
      start(c0 + 2, hb0, s0)
    wait(c0 + 1, hb1, s1)
    compute_chunk(hb1, c0 + 1)

  pltpu.sync_copy(ov, out_hbm.at[pl.ds(wid * RWS, RWS)])


def _tc_body(x_ref, h_ref, w_ref, b_ref, out_ref):
  hv = h_ref[...]
  wv = w_ref[...]
  dots = lax.dot_general(hv, wv, (((1,), (0,)), ((), ())),
                         preferred_element_type=jnp.float32)
  maskf = jnp.where(x_ref[...] == CLF_TOKEN, jnp.float32(1.0),
                    jnp.float32(0.0))
  out_ref[...] = dots * maskf + b_ref[0]


@jax.jit
def _clf_head(h, xt, wf, bf):
  mesh = plsc.VectorSubcoreMesh(core_axis_name="c", subcore_axis_name="s",
                                num_cores=NC, num_subcores=NS)
  sc_fn = pl.kernel(
      _sc_body,
      out_type=jax.ShapeDtypeStruct((SC_ROWS,), jnp.float32),
      mesh=mesh,
      scratch_types=[
          pltpu.VMEM((C, N_EMBED), jnp.float32),
          pltpu.VMEM((C, N_EMBED), jnp.float32),
          pltpu.VMEM((N_EMBED,), jnp.float32),
          pltpu.VMEM((RWS,), jnp.int32),
          pltpu.VMEM((LANES,), jnp.float32),
          pltpu.VMEM((RWS,), jnp.float32),
          pltpu.SemaphoreType.DMA,
          pltpu.SemaphoreType.DMA,
      ],
  )
  out_sc = sc_fn(h, xt, wf, bf)

  hf = h.reshape(ROWS, N_EMBED)
  xf = xt.reshape(ROWS)
  out_tc = pl.pallas_call(
      _tc_body,
      grid=(RT // BR,),
      in_specs=[
          pl.BlockSpec((BR,), lambda i: (i,)),
          pl.BlockSpec((BR, N_EMBED), lambda i: (i, 0)),
          pl.BlockSpec((N_EMBED,), lambda i: (0,)),
          pl.BlockSpec((LANES,), lambda i: (0,)),
      ],
      out_specs=pl.BlockSpec((BR,), lambda i: (i,)),
      out_shape=jax.ShapeDtypeStruct((RT,), jnp.float32),
  )(xf, hf, wf, bf)

  return jnp.concatenate([out_tc, out_sc]).reshape(-1, 2)


def kernel(h, x, W, b):
  xt = x[:, :, 0]
  wf = W.reshape(N_EMBED).astype(jnp.float32)
  bf = jnp.tile(b.astype(jnp.float32), LANES)
  return _clf_head(h, xt, wf, bf)


# speedup vs baseline: 1.1283x; 1.1283x over previous
import jax
import jax.numpy as jnp
from jax import lax
from jax.experimental import pallas as pl
from jax.experimental.pallas import tpu as pltpu
from jax.experimental.pallas import tpu_sc as plsc

N_EMBED = 768
CLF_TOKEN = 0
ROWS = 16384
SEQ = 2048
NC, NS = 2, 16
NW = NC * NS
RWS = 256
SC_ROWS = NW * RWS
RT = ROWS - SC_ROWS
C = 64
NCH = RWS // C
NP = NCH // 2
LANES = 16
NJ = N_EMBED // LANES
BR = 2048


def _sc_body(h_hbm, x_hbm, w_hbm, b_hbm, out_hbm,
             hb0, hb1, wv, xv, bv, ov, s0, s1, s2):
  cid = lax.axis_index("c")
  sid = lax.axis_index("s")
  wid = sid * NC + cid
  gbase = RT + wid * RWS
  bidx = gbase // SEQ
  roff = gbase % SEQ

  pltpu.async_copy(h_hbm.at[bidx, pl.ds(roff, C), :], hb0, s0)
  cw = pltpu.async_copy(w_hbm.at[0], wv, s2)
  cb = pltpu.async_copy(b_hbm, bv.at[pl.ds(0, 1)], s2)
  cx = pltpu.async_copy(x_hbm.at[bidx, pl.ds(roff, RWS)], xv, s2)
  cw.wait()
  cb.wait()
  cx.wait()

  lane = lax.broadcasted_iota(jnp.int32, (LANES,), 0)
  perms = [lane ^ sh for sh in (8, 4, 2, 1)]

  def lane_sum(v):
    for perm in perms:
      v = v + jnp.take_along_axis(v, perm, axis=0, mode="promise_in_bounds")
    return v

  zero16 = jnp.zeros((LANES,), jnp.float32)
  lane0 = jnp.zeros((LANES,), jnp.int32)

  bsplat = jnp.take_along_axis(bv[...], lane0, axis=0,
                               mode="promise_in_bounds")

  def compute_chunk(buf, chunk_idx):
    def group(gi, carry):
      rr = gi * LANES
      @plsc.parallel_loop(0, NJ, unroll=2,
                          carry=tuple(zero16 for _ in range(LANES)))
      def accs(j, acc):
        off = j * LANES
        wj = wv[pl.ds(off, LANES)]
        return tuple(
            acc[i] + buf[rr + i, pl.ds(off, LANES)] * wj
            for i in range(LANES))

      out_vec = lane_sum(accs[0])
      for i in range(1, LANES):
        out_vec = jnp.where(lane == i, lane_sum(accs[i]), out_vec)
      rloc = chunk_idx * C + rr
      tokv = xv[pl.ds(rloc, LANES)]
      maskv = jnp.where(tokv == CLF_TOKEN, jnp.float32(1.0),
                        jnp.float32(0.0))
      res = out_vec * maskv + bsplat
      ov[pl.ds(rloc, LANES)] = res
      return carry

    lax.fori_loop(0, C // LANES, group, 0)

  def start(chunk, buf, sem):
    pltpu.async_copy(h_hbm.at[bidx, pl.ds(roff + chunk * C, C), :], buf, sem)

  def wait(chunk, buf, sem):
    pltpu.make_async_copy(h_hbm.at[bidx, pl.ds(roff + chunk * C, C), :], buf,
                          sem).wait()

  for p in range(NP):
    c0 = p * 2
    start(c0 + 1, hb1, s1)
    wait(c0, hb0, s0)
    compute_chunk(hb0, c0)
    if p < NP - 1:
      start(c0 + 2, hb0, s0)
    wait(c0 + 1, hb1, s1)
    compute_chunk(hb1, c0 + 1)

  pltpu.sync_copy(ov, out_hbm.at[pl.ds(wid * RWS, RWS)])


def _tc_body(x_ref, h_ref, w_ref, b_ref, out_ref):
  hv = h_ref[...]
  wv = w_ref[0]
  dots = lax.dot_general(hv, wv, (((1,), (0,)), ((), ())),
                         preferred_element_type=jnp.float32)
  maskf = jnp.where(x_ref[...] == CLF_TOKEN, jnp.float32(1.0),
                    jnp.float32(0.0))
  out_ref[...] = dots * maskf + b_ref[0]


@jax.jit
def _clf_head(h, xt, wf, bf):
  mesh = plsc.VectorSubcoreMesh(core_axis_name="c", subcore_axis_name="s",
                                num_cores=NC, num_subcores=NS)
  sc_fn = pl.kernel(
      _sc_body,
      out_type=jax.ShapeDtypeStruct((SC_ROWS,), jnp.float32),
      mesh=mesh,
      scratch_types=[
          pltpu.VMEM((C, N_EMBED), jnp.float32),
          pltpu.VMEM((C, N_EMBED), jnp.float32),
          pltpu.VMEM((N_EMBED,), jnp.float32),
          pltpu.VMEM((RWS,), jnp.int32),
          pltpu.VMEM((LANES,), jnp.float32),
          pltpu.VMEM((RWS,), jnp.float32),
          pltpu.SemaphoreType.DMA,
          pltpu.SemaphoreType.DMA,
          pltpu.SemaphoreType.DMA,
      ],
  )
  out_sc = sc_fn(h, xt, wf, bf)

  hf = h.reshape(ROWS, N_EMBED)
  xf = xt.reshape(ROWS)
  out_tc = pl.pallas_call(
      _tc_body,
      grid=(RT // BR,),
      in_specs=[
          pl.BlockSpec((BR,), lambda i: (i,)),
          pl.BlockSpec((BR, N_EMBED), lambda i: (i, 0)),
          pl.BlockSpec((1, N_EMBED), lambda i: (0, 0)),
          pl.BlockSpec((1,), lambda i: (0,)),
      ],
      out_specs=pl.BlockSpec((BR,), lambda i: (i,)),
      out_shape=jax.ShapeDtypeStruct((RT,), jnp.float32),
  )(xf, hf, wf, bf)

  return jnp.concatenate([out_tc, out_sc]).reshape(-1, 2)


def kernel(h, x, W, b):
  xt = x[:, :, 0]
  return _clf_head(h, xt, W, b)
